# Initial kernel scaffold; baseline (speedup 1.0000x reference)
import functools

import jax
import jax.numpy as jnp
from jax import lax
from jax.experimental import pallas as pl
from jax.experimental.pallas import tpu as pltpu
from jax.experimental.pallas import tpu_sc as plsc

N = 10000
E = 320000
D = 128

NC = 2
NS = 16
NW = NC * NS
CHUNK = 128
ROWS = 80
EPAD = NW * ROWS * CHUNK
NPAD = 10240
ROWS_S = N // NS
ROWS_C = NPAD // NS


def _sc_body(src_h, dst_h, x_h, zs_h, zc_h, ones_h, psum_h, pcnt_h,
             src_v, dst_v, buf0, buf1, ones_v, acc_s, acc_c, sem0, sem1):
    cid = lax.axis_index("c")
    sid = lax.axis_index("s")
    wid = sid * NC + cid

    pltpu.sync_copy(zs_h, acc_s.at[pl.ds(sid * ROWS_C, ROWS_C)])
    pltpu.sync_copy(zc_h, acc_c.at[pl.ds(sid * ROWS_C, ROWS_C)])
    pltpu.sync_copy(ones_h, ones_v)
    pltpu.sync_copy(src_h.at[wid], src_v)
    pltpu.sync_copy(dst_h.at[wid], dst_v)
    plsc.subcore_barrier()

    cp0 = pltpu.async_copy(x_h.at[src_v.at[0]], buf0, sem0)
    cp1 = pltpu.async_copy(x_h.at[src_v.at[1]], buf1, sem1)

    def step(g, carry):
        for b, buf, sem, cp in ((0, buf0, sem0, cp0), (1, buf1, sem1, cp1)):
            j = g + b
            cp.wait()
            pltpu.sync_copy(buf, acc_s.at[dst_v.at[j]], add=True)
            pltpu.sync_copy(ones_v, acc_c.at[dst_v.at[j]], add=True)

            @pl.when(j + 2 < ROWS)
            def _():
                pltpu.async_copy(x_h.at[src_v.at[j + 2]], buf, sem)

        return carry

    lax.fori_loop(0, ROWS // 2, lambda i, c: step(i * 2, c), 0)
    plsc.subcore_barrier()

    pltpu.sync_copy(acc_s.at[pl.ds(sid * ROWS_S, ROWS_S)],
                    psum_h.at[cid, pl.ds(sid * ROWS_S, ROWS_S)])
    pltpu.sync_copy(acc_c.at[pl.ds(sid * ROWS_C, ROWS_C)],
                    pcnt_h.at[cid, pl.ds(sid * ROWS_C, ROWS_C)])


_sc_call = functools.partial(
    pl.kernel,
    out_type=[
        jax.ShapeDtypeStruct((NC, N, D), jnp.float32),
        jax.ShapeDtypeStruct((NC, NPAD, 16), jnp.float32),
    ],
    mesh=plsc.VectorSubcoreMesh(core_axis_name="c", subcore_axis_name="s"),
    scratch_types=[
        pltpu.VMEM((ROWS, CHUNK), jnp.int32),
        pltpu.VMEM((ROWS, CHUNK), jnp.int32),
        pltpu.VMEM((CHUNK, D), jnp.float32),
        pltpu.VMEM((CHUNK, D), jnp.float32),
        pltpu.VMEM((CHUNK, 16), jnp.float32),
        pltpu.VMEM_SHARED((NPAD, D), jnp.float32),
        pltpu.VMEM_SHARED((NPAD, 16), jnp.float32),
        pltpu.SemaphoreType.DMA,
        pltpu.SemaphoreType.DMA,
    ],
)(_sc_body)


def _tc_body(psum_ref, pcnt_ref, x_ref, wlt_ref, bl_ref, wrt_ref, g_ref, b_ref,
             out_ref):
    s = psum_ref[0] + psum_ref[1]
    c = pcnt_ref[0, 0:N, 0:1] + pcnt_ref[1, 0:N, 0:1]
    mean = s / jnp.maximum(c, 1.0)
    h = (jnp.dot(mean, wlt_ref[...], preferred_element_type=jnp.float32)
         + jnp.dot(x_ref[...], wrt_ref[...], preferred_element_type=jnp.float32)
         + bl_ref[...][None, :])
    h = jnp.maximum(h, 0.0)
    mu = jnp.mean(h, axis=0, keepdims=True)
    d = h - mu
    var = jnp.mean(d * d, axis=0, keepdims=True)
    out_ref[...] = (d * lax.rsqrt(var + 1e-5) * g_ref[...][None, :]
                    + b_ref[...][None, :])


_tc_call = pl.pallas_call(
    _tc_body,
    out_shape=jax.ShapeDtypeStruct((N, D), jnp.float32),
)


@jax.jit
def kernel(x, edge_index, W_l, b_l, W_r, gamma, beta):
    src = edge_index[0]
    dst = edge_index[1]
    pad = EPAD - E
    src3 = jnp.concatenate([src, jnp.zeros((pad,), jnp.int32)]).reshape(
        NW, ROWS, CHUNK)
    dst3 = jnp.concatenate([dst, jnp.full((pad,), N, jnp.int32)]).reshape(
        NW, ROWS, CHUNK)
    zs = jnp.zeros((ROWS_C, D), jnp.float32)
    zc = jnp.zeros((ROWS_C, 16), jnp.float32)
    ones = jnp.ones((CHUNK, 16), jnp.float32)
    psum, pcnt = _sc_call(src3, dst3, x, zs, zc, ones)
    return _tc_call(psum, pcnt, x, W_l.T, b_l, W_r.T, gamma, beta)


# speedup vs baseline: 5.9789x; 5.9789x over previous
import functools

import jax
import jax.numpy as jnp
from jax import lax
from jax.experimental import pallas as pl
from jax.experimental.pallas import tpu as pltpu
from jax.experimental.pallas import tpu_sc as plsc

N = 10000
E = 320000
D = 128
DH = D // 2

NC = 2
NS = 16
CHUNK = 128
ROWS = 160
EPAD = NS * ROWS * CHUNK
NPAD = 10240
ROWS_T = NPAD // NS


def _sc_body(src_h, dst_h, xh_h, zs_h, zc_h, ones_h, psum_h, pcnt_h,
             src_v, dst_v, buf0, buf1, ones_v, acc_s, acc_c, sem0, sem1):
    cid = lax.axis_index("c")
    sid = lax.axis_index("s")
    x_half = xh_h.at[cid]

    pltpu.sync_copy(zs_h, acc_s.at[pl.ds(sid * ROWS_T, ROWS_T)])
    pltpu.sync_copy(src_h.at[sid], src_v)
    pltpu.sync_copy(dst_h.at[sid], dst_v)

    @pl.when(cid == 0)
    def _():
        pltpu.sync_copy(zc_h, acc_c.at[pl.ds(sid * ROWS_T, ROWS_T)])
        pltpu.sync_copy(ones_h, ones_v)

    plsc.subcore_barrier()

    cp0 = pltpu.async_copy(x_half.at[src_v.at[0]], buf0, sem0)
    cp1 = pltpu.async_copy(x_half.at[src_v.at[1]], buf1, sem1)

    def step(g, carry):
        for b, buf, sem, cp in ((0, buf0, sem0, cp0), (1, buf1, sem1, cp1)):
            j = g + b
            cp.wait()
            pltpu.sync_copy(buf, acc_s.at[dst_v.at[j]], add=True)

            @pl.when(cid == 0)
            def _():
                pltpu.sync_copy(ones_v, acc_c.at[dst_v.at[j]], add=True)

            @pl.when(j + 2 < ROWS)
            def _():
                pltpu.async_copy(x_half.at[src_v.at[j + 2]], buf, sem)

        return carry

    lax.fori_loop(0, ROWS // 2, lambda i, c: step(i * 2, c), 0)
    plsc.subcore_barrier()

    pltpu.sync_copy(acc_s.at[pl.ds(sid * ROWS_T, ROWS_T)],
                    psum_h.at[cid, pl.ds(sid * ROWS_T, ROWS_T)])

    @pl.when(cid == 0)
    def _():
        pltpu.sync_copy(acc_c.at[pl.ds(sid * ROWS_T, ROWS_T)],
                        pcnt_h.at[pl.ds(sid * ROWS_T, ROWS_T)])


_sc_call = functools.partial(
    pl.kernel,
    out_type=[
        jax.ShapeDtypeStruct((NC, NPAD, DH), jnp.float32),
        jax.ShapeDtypeStruct((NPAD, 16), jnp.float32),
    ],
    mesh=plsc.VectorSubcoreMesh(core_axis_name="c", subcore_axis_name="s"),
    compiler_params=pltpu.CompilerParams(use_tc_tiling_on_sc=False),
    scratch_types=[
        pltpu.VMEM((ROWS, CHUNK), jnp.int32),
        pltpu.VMEM((ROWS, CHUNK), jnp.int32),
        pltpu.VMEM((CHUNK, DH), jnp.float32),
        pltpu.VMEM((CHUNK, DH), jnp.float32),
        pltpu.VMEM((CHUNK, 16), jnp.float32),
        pltpu.VMEM_SHARED((NPAD, DH), jnp.float32),
        pltpu.VMEM_SHARED((NPAD, 16), jnp.float32),
        pltpu.SemaphoreType.DMA,
        pltpu.SemaphoreType.DMA,
    ],
)(_sc_body)


def _tc_body(psum_ref, pcnt_ref, x_ref, wlt_ref, bl_ref, wrt_ref, g_ref, b_ref,
             out_ref):
    s = jnp.concatenate([psum_ref[0, 0:N, :], psum_ref[1, 0:N, :]], axis=1)
    c = pcnt_ref[0:N, 0:1]
    mean = s / jnp.maximum(c, 1.0)
    h = (jnp.dot(mean, wlt_ref[...], preferred_element_type=jnp.float32)
         + jnp.dot(x_ref[...], wrt_ref[...], preferred_element_type=jnp.float32)
         + bl_ref[...][None, :])
    h = jnp.maximum(h, 0.0)
    mu = jnp.mean(h, axis=0, keepdims=True)
    d = h - mu
    var = jnp.mean(d * d, axis=0, keepdims=True)
    out_ref[...] = (d * lax.rsqrt(var + 1e-5) * g_ref[...][None, :]
                    + b_ref[...][None, :])


_tc_call = pl.pallas_call(
    _tc_body,
    out_shape=jax.ShapeDtypeStruct((N, D), jnp.float32),
)


@jax.jit
def kernel(x, edge_index, W_l, b_l, W_r, gamma, beta):
    src = edge_index[0]
    dst = edge_index[1]
    pad = EPAD - E
    src3 = jnp.concatenate([src, jnp.zeros((pad,), jnp.int32)]).reshape(
        NS, ROWS, CHUNK)
    dst3 = jnp.concatenate([dst, jnp.full((pad,), N, jnp.int32)]).reshape(
        NS, ROWS, CHUNK)
    xh = x.reshape(N, NC, DH).transpose(1, 0, 2)
    zs = jnp.zeros((ROWS_T, DH), jnp.float32)
    zc = jnp.zeros((ROWS_T, 16), jnp.float32)
    ones = jnp.ones((CHUNK, 16), jnp.float32)
    psum, pcnt = _sc_call(src3, dst3, xh, zs, zc, ones)
    return _tc_call(psum, pcnt, x, W_l.T, b_l, W_r.T, gamma, beta)
